# baseline (speedup 1.0000x reference)
import functools

import jax
import jax.numpy as jnp
from jax import lax
from jax.experimental import pallas as pl
from jax.experimental.pallas import tpu as pltpu
from jax.experimental.pallas import tpu_sc as plsc

N = 16384
B = 8
F = 128
R = 0.15
K = 32
NW = 32
LANES = 16
QT = 128
PT = 1024


def _pre_body(x_ref, p8_ref, w1a_ref, w1b_ref, b1_ref, u2_ref, t_ref):
    t = jnp.dot(p8_ref[...], w1b_ref[...], preferred_element_type=jnp.float32)
    u2_ref[...] = (
        jnp.dot(x_ref[...], w1a_ref[...], preferred_element_type=jnp.float32)
        + t + b1_ref[...]
    )
    t_ref[...] = t


def _precompute(x, pos8, w1a, w1b8, b1r):
    n, f = x.shape
    h = w1a.shape[1]
    return pl.pallas_call(
        _pre_body,
        grid=(n // PT,),
        in_specs=[
            pl.BlockSpec((PT, f), lambda i: (i, 0)),
            pl.BlockSpec((PT, 8), lambda i: (i, 0)),
            pl.BlockSpec((f, h), lambda i: (0, 0)),
            pl.BlockSpec((8, h), lambda i: (0, 0)),
            pl.BlockSpec((1, h), lambda i: (0, 0)),
        ],
        out_specs=[
            pl.BlockSpec((PT, h), lambda i: (i, 0)),
            pl.BlockSpec((PT, h), lambda i: (i, 0)),
        ],
        out_shape=[
            jax.ShapeDtypeStruct((n, h), jnp.float32),
            jax.ShapeDtypeStruct((n, h), jnp.float32),
        ],
    )(x, pos8, w1a, w1b8, b1r)


def _sc_body(posx_hbm, posy_hbm, posz_hbm, lo_hbm, hi_hbm, u2_hbm, ug_hbm,
             px_v, py_v, pz_v, lo_v, hi_v, colg_v, rows_v, semg, semw):
    n = px_v.shape[0]
    qpw = lo_v.shape[0]
    epg = rows_v.shape[0]
    k = epg // LANES
    gpw = qpw // LANES
    nchunk = epg // 128

    cid = lax.axis_index("c")
    sid = lax.axis_index("s")
    wid = sid * 2 + cid
    wbase = wid * qpw

    pltpu.sync_copy(posx_hbm, px_v)
    pltpu.sync_copy(posy_hbm, py_v)
    pltpu.sync_copy(posz_hbm, pz_v)
    pltpu.sync_copy(lo_hbm.at[pl.ds(wbase, qpw)], lo_v)
    pltpu.sync_copy(hi_hbm.at[pl.ds(wbase, qpw)], hi_v)

    lanes = lax.iota(jnp.int32, LANES)
    r2 = jnp.full((LANES,), R * R, jnp.float32)

    def search_group(g, off):
        qg = wbase + g * LANES
        qx = px_v[pl.ds(qg, LANES)]
        qy = py_v[pl.ds(qg, LANES)]
        qz = pz_v[pl.ds(qg, LANES)]
        qlo = lo_v[pl.ds(g * LANES, LANES)]
        qhi = hi_v[pl.ds(g * LANES, LANES)]
        glo_c = qlo[0] >> 4
        ghi_c = (qhi[LANES - 1] + LANES - 1) >> 4
        ms = jnp.maximum(glo_c, jnp.minimum((qlo[LANES - 1] + LANES - 1) >> 4,
                                            qhi[0] >> 4))
        me = jnp.maximum(ms, qhi[0] >> 4)

        selfidx = qg + lanes
        for kk in range(k):
            plsc.store_scatter(colg_v, [off + lanes * k + kk], selfidx)

        lanes_k = off + lanes * k

        def make_chunk_body(checked):
            def cand_chunk(c, cnt):
                base = c * LANES
                cx = px_v[pl.ds(base, LANES)]
                cy = py_v[pl.ds(base, LANES)]
                cz = pz_v[pl.ds(base, LANES)]
                for l in range(LANES):
                    j = base + l
                    dx = qx - cx[l]
                    dy = qy - cy[l]
                    dz = qz - cz[l]
                    d2 = dx * dx + dy * dy + dz * dz
                    m = (d2 <= r2) & (cnt < k)
                    if checked:
                        m = m & (j >= qlo) & (j < qhi)
                    plsc.store_scatter(colg_v, [lanes_k + cnt],
                                       jnp.full((LANES,), j, jnp.int32),
                                       mask=m)
                    cnt = cnt + m.astype(jnp.int32)
                return cnt
            return cand_chunk

        cnt = lax.fori_loop(glo_c, ms, make_chunk_body(True),
                            jnp.zeros((LANES,), jnp.int32))
        cnt = lax.fori_loop(ms, me, make_chunk_body(False), cnt)
        lax.fori_loop(me, ghi_c, make_chunk_body(True), cnt)

    def gather_slices(off, c):
        idx = colg_v.at[pl.ds(off + c * 128, 128)]
        dst = rows_v.at[pl.ds(c * 128, 128)]
        return u2_hbm.at[idx], dst

    def issue_gathers(off):
        for c in range(nchunk):
            src, dst = gather_slices(off, c)
            pltpu.async_copy(src, dst, semg)

    def wait_gathers(off):
        for c in range(nchunk):
            src, dst = gather_slices(off, c)
            pltpu.make_async_copy(src, dst, semg).wait()

    def write_slices(g):
        dst = ug_hbm.at[pl.ds((wbase + g * LANES) * k, epg)]
        return rows_v, dst

    def issue_write(g):
        src, dst = write_slices(g)
        pltpu.async_copy(src, dst, semw)

    def wait_write(g):
        src, dst = write_slices(g)
        pltpu.make_async_copy(src, dst, semw).wait()

    def group_body(g, _):
        off = (g & 1) * epg
        search_group(g, off)

        @pl.when(g >= 1)
        def _():
            oth = epg - off
            wait_gathers(oth)
            issue_write(g - 1)
            wait_write(g - 1)

        issue_gathers(off)
        return 0

    lax.fori_loop(0, gpw, group_body, 0)
    wait_gathers(((gpw - 1) & 1) * epg)
    issue_write(gpw - 1)
    wait_write(gpw - 1)


def _sc_search_gather(posx, posy, posz, lo, hi, u2):
    n = posx.shape[0]
    h = u2.shape[1]
    qpw = n // NW
    epg = LANES * K
    mesh = plsc.VectorSubcoreMesh(core_axis_name="c", subcore_axis_name="s",
                                  num_cores=2, num_subcores=16)
    return pl.kernel(
        _sc_body,
        out_type=jax.ShapeDtypeStruct((n * K, h), jnp.float32),
        mesh=mesh,
        compiler_params=pltpu.CompilerParams(needs_layout_passes=False),
        scratch_types=[
            pltpu.VMEM((n,), jnp.float32),
            pltpu.VMEM((n,), jnp.float32),
            pltpu.VMEM((n,), jnp.float32),
            pltpu.VMEM((qpw,), jnp.int32),
            pltpu.VMEM((qpw,), jnp.int32),
            pltpu.VMEM((2 * epg,), jnp.int32),
            pltpu.VMEM((epg, h), jnp.float32),
            pltpu.SemaphoreType.DMA,
            pltpu.SemaphoreType.DMA,
        ],
    )(posx, posy, posz, lo, hi, u2)


def _mlp_body(ug_ref, t_ref, w2_ref, b2_ref, w3_ref, b3_ref, out_ref):
    qt = t_ref.shape[0]
    hid = w2_ref.shape[0]
    ho = w3_ref.shape[1]
    ug = ug_ref[...].reshape(qt, K, 2 * hid)[:, :, :hid]
    h1 = jnp.maximum(ug - t_ref[...][:, None, :hid], 0.0).reshape(qt * K, hid)
    h2 = jnp.maximum(
        jnp.dot(h1.astype(jnp.bfloat16), w2_ref[...].astype(jnp.bfloat16),
                preferred_element_type=jnp.float32) + b2_ref[...], 0.0)
    h3 = jnp.maximum(
        jnp.dot(h2.astype(jnp.bfloat16), w3_ref[...].astype(jnp.bfloat16),
                preferred_element_type=jnp.float32) + b3_ref[...], 0.0)
    out_ref[...] = jnp.max(h3.reshape(qt, K, ho), axis=1)


def _mlp(ug, t, w2, b2r, w3, b3r):
    n = t.shape[0]
    hid = w2.shape[0]
    ho = w3.shape[1]
    return pl.pallas_call(
        _mlp_body,
        grid=(n // QT,),
        in_specs=[
            pl.BlockSpec((QT * K, 2 * hid), lambda i: (i, 0)),
            pl.BlockSpec((QT, 2 * hid), lambda i: (i, 0)),
            pl.BlockSpec((hid, hid), lambda i: (0, 0)),
            pl.BlockSpec((1, hid), lambda i: (0, 0)),
            pl.BlockSpec((hid, ho), lambda i: (0, 0)),
            pl.BlockSpec((1, ho), lambda i: (0, 0)),
        ],
        out_specs=pl.BlockSpec((QT, ho), lambda i: (i, 0)),
        out_shape=jax.ShapeDtypeStruct((n, ho), jnp.float32),
    )(ug, t, w2, b2r, w3, b3r)


def kernel(x, pos, batch, W1, b1, W2, b2, W3, b3):
    n, f = x.shape
    bi = batch.astype(jnp.int32)
    offs = jnp.searchsorted(
        bi, jnp.arange(B + 1, dtype=jnp.int32), side="left").astype(jnp.int32)
    lo = offs[bi]
    hi = offs[bi + 1]
    posx = pos[:, 0]
    posy = pos[:, 1]
    posz = pos[:, 2]
    pos8 = jnp.pad(pos, ((0, 0), (0, 5)))
    hp = F - 64
    w1a = jnp.pad(W1[:f], ((0, 0), (0, hp)))
    w1b8 = jnp.pad(W1[f:], ((0, 5), (0, hp)))
    b1r = jnp.pad(b1, (0, hp)).reshape(1, -1)

    u2, t = _precompute(x, pos8, w1a, w1b8, b1r)
    ug = _sc_search_gather(posx, posy, posz, lo, hi, u2)
    out = _mlp(ug, t, W2, b2.reshape(1, -1), W3, b3.reshape(1, -1))
    return (out, pos, batch)


# speedup vs baseline: 1.0673x; 1.0673x over previous
import functools

import jax
import jax.numpy as jnp
from jax import lax
from jax.experimental import pallas as pl
from jax.experimental.pallas import tpu as pltpu
from jax.experimental.pallas import tpu_sc as plsc

N = 16384
B = 8
F = 128
R = 0.15
K = 32
NW = 32
LANES = 16
QT = 128
PT = 1024


def _pre_body(x_ref, p8_ref, w1a_ref, w1bp_ref, w1b_ref, b1_ref,
              u2_ref, t_ref):
    u2_ref[...] = (
        jnp.dot(x_ref[...], w1a_ref[...], preferred_element_type=jnp.float32)
        + jnp.dot(p8_ref[...], w1bp_ref[...],
                  preferred_element_type=jnp.float32)
        + b1_ref[...]
    )
    t_ref[...] = jnp.dot(p8_ref[...], w1b_ref[...],
                         preferred_element_type=jnp.float32)


def _precompute(x, pos8, w1a, w1b8p, w1b8, b1r):
    n, f = x.shape
    h = w1a.shape[1]
    hid = w1b8.shape[1]
    return pl.pallas_call(
        _pre_body,
        grid=(n // PT,),
        in_specs=[
            pl.BlockSpec((PT, f), lambda i: (i, 0)),
            pl.BlockSpec((PT, 8), lambda i: (i, 0)),
            pl.BlockSpec((f, h), lambda i: (0, 0)),
            pl.BlockSpec((8, h), lambda i: (0, 0)),
            pl.BlockSpec((8, hid), lambda i: (0, 0)),
            pl.BlockSpec((1, h), lambda i: (0, 0)),
        ],
        out_specs=[
            pl.BlockSpec((PT, h), lambda i: (i, 0)),
            pl.BlockSpec((PT, hid), lambda i: (i, 0)),
        ],
        out_shape=[
            jax.ShapeDtypeStruct((n, h), jnp.float32),
            jax.ShapeDtypeStruct((n, hid), jnp.float32),
        ],
    )(x, pos8, w1a, w1b8p, w1b8, b1r)


def _make_sc_body(qbase):
  def _sc_body(posx_hbm, posy_hbm, posz_hbm, lo_hbm, hi_hbm, u2_hbm, ug_hbm,
               px_v, py_v, pz_v, lo_v, hi_v, colg_v, rows_v, semg, semw):
    n = px_v.shape[0]
    qpw = lo_v.shape[0]
    epg = rows_v.shape[0]
    k = epg // LANES
    gpw = qpw // LANES
    nchunk = epg // 128

    cid = lax.axis_index("c")
    sid = lax.axis_index("s")
    wid = sid * 2 + cid
    wloc = wid * qpw
    wbase = qbase + wloc

    pltpu.sync_copy(posx_hbm, px_v)
    pltpu.sync_copy(posy_hbm, py_v)
    pltpu.sync_copy(posz_hbm, pz_v)
    pltpu.sync_copy(lo_hbm.at[pl.ds(wbase, qpw)], lo_v)
    pltpu.sync_copy(hi_hbm.at[pl.ds(wbase, qpw)], hi_v)

    lanes = lax.iota(jnp.int32, LANES)
    r2 = jnp.full((LANES,), R * R, jnp.float32)

    def search_group(g, off):
        qg = wbase + g * LANES
        qx = px_v[pl.ds(qg, LANES)]
        qy = py_v[pl.ds(qg, LANES)]
        qz = pz_v[pl.ds(qg, LANES)]
        qlo = lo_v[pl.ds(g * LANES, LANES)]
        qhi = hi_v[pl.ds(g * LANES, LANES)]
        glo_c = qlo[0] >> 4
        ghi_c = (qhi[LANES - 1] + LANES - 1) >> 4
        ms = jnp.maximum(glo_c, jnp.minimum((qlo[LANES - 1] + LANES - 1) >> 4,
                                            qhi[0] >> 4))
        me = jnp.maximum(ms, qhi[0] >> 4)

        selfidx = qg + lanes
        for kk in range(k):
            plsc.store_scatter(colg_v, [off + lanes * k + kk], selfidx)

        lanes_k = off + lanes * k

        def make_chunk_body(checked):
            def cand_chunk(c, cnt):
                base = c * LANES
                cx = px_v[pl.ds(base, LANES)]
                cy = py_v[pl.ds(base, LANES)]
                cz = pz_v[pl.ds(base, LANES)]
                for l in range(LANES):
                    j = base + l
                    dx = qx - cx[l]
                    dy = qy - cy[l]
                    dz = qz - cz[l]
                    d2 = dx * dx + dy * dy + dz * dz
                    m = (d2 <= r2) & (cnt < k)
                    if checked:
                        m = m & (j >= qlo) & (j < qhi)
                    plsc.store_scatter(colg_v, [lanes_k + cnt],
                                       jnp.full((LANES,), j, jnp.int32),
                                       mask=m)
                    cnt = cnt + m.astype(jnp.int32)
                return cnt
            return cand_chunk

        cnt = lax.fori_loop(glo_c, ms, make_chunk_body(True),
                            jnp.zeros((LANES,), jnp.int32))
        cnt = lax.fori_loop(ms, me, make_chunk_body(False), cnt)
        lax.fori_loop(me, ghi_c, make_chunk_body(True), cnt)

    def gather_slices(off, c):
        idx = colg_v.at[pl.ds(off + c * 128, 128)]
        dst = rows_v.at[pl.ds(c * 128, 128)]
        return u2_hbm.at[idx], dst

    def issue_gathers(off):
        for c in range(nchunk):
            src, dst = gather_slices(off, c)
            pltpu.async_copy(src, dst, semg)

    def wait_gathers(off):
        for c in range(nchunk):
            src, dst = gather_slices(off, c)
            pltpu.make_async_copy(src, dst, semg).wait()

    def write_slices(g):
        dst = ug_hbm.at[pl.ds((wloc + g * LANES) * k, epg)]
        return rows_v, dst

    def issue_write(g):
        src, dst = write_slices(g)
        pltpu.async_copy(src, dst, semw)

    def wait_write(g):
        src, dst = write_slices(g)
        pltpu.make_async_copy(src, dst, semw).wait()

    def group_body(g, _):
        off = (g & 1) * epg
        search_group(g, off)

        @pl.when(g >= 1)
        def _():
            wait_gathers(epg - off)
            issue_write(g - 1)
            wait_write(g - 1)

        issue_gathers(off)
        return 0

    lax.fori_loop(0, gpw, group_body, 0)
    wait_gathers(((gpw - 1) & 1) * epg)
    issue_write(gpw - 1)
    wait_write(gpw - 1)
  return _sc_body


def _sc_search_gather(posx, posy, posz, lo, hi, u2, qbase, nq):
    n = posx.shape[0]
    h = u2.shape[1]
    qpw = nq // NW
    epg = LANES * K
    mesh = plsc.VectorSubcoreMesh(core_axis_name="c", subcore_axis_name="s",
                                  num_cores=2, num_subcores=16)
    return pl.kernel(
        _make_sc_body(qbase),
        out_type=jax.ShapeDtypeStruct((nq * K, h), jnp.float32),
        mesh=mesh,
        compiler_params=pltpu.CompilerParams(needs_layout_passes=False),
        scratch_types=[
            pltpu.VMEM((n,), jnp.float32),
            pltpu.VMEM((n,), jnp.float32),
            pltpu.VMEM((n,), jnp.float32),
            pltpu.VMEM((qpw,), jnp.int32),
            pltpu.VMEM((qpw,), jnp.int32),
            pltpu.VMEM((2 * epg,), jnp.int32),
            pltpu.VMEM((epg, h), jnp.float32),
            pltpu.SemaphoreType.DMA,
            pltpu.SemaphoreType.DMA,
        ],
    )(posx, posy, posz, lo, hi, u2)


def _mlp_body(ug_ref, t_ref, w2_ref, b2_ref, w3_ref, b3_ref, out_ref):
    qt = t_ref.shape[0]
    hid = w2_ref.shape[0]
    ho = w3_ref.shape[1]
    ug = ug_ref[...][:, :hid].reshape(qt, K, hid)
    h1 = jnp.maximum(ug - t_ref[...][:, None, :], 0.0).reshape(qt * K, hid)
    h2 = jnp.maximum(
        jnp.dot(h1.astype(jnp.bfloat16), w2_ref[...].astype(jnp.bfloat16),
                preferred_element_type=jnp.float32) + b2_ref[...], 0.0)
    h3 = jnp.maximum(
        jnp.dot(h2.astype(jnp.bfloat16), w3_ref[...].astype(jnp.bfloat16),
                preferred_element_type=jnp.float32) + b3_ref[...], 0.0)
    out_ref[...] = jnp.max(h3.reshape(qt, K, ho), axis=1)


def _mlp(ug, t, w2, b2r, w3, b3r, qoff):
    nq = ug.shape[0] // K
    hid = w2.shape[0]
    ho = w3.shape[1]
    return pl.pallas_call(
        _mlp_body,
        grid=(nq // QT,),
        in_specs=[
            pl.BlockSpec((QT * K, 2 * hid), lambda i: (i, 0)),
            pl.BlockSpec((QT, hid), lambda i, qo=qoff: (i + qo, 0)),
            pl.BlockSpec((hid, hid), lambda i: (0, 0)),
            pl.BlockSpec((1, hid), lambda i: (0, 0)),
            pl.BlockSpec((hid, ho), lambda i: (0, 0)),
            pl.BlockSpec((1, ho), lambda i: (0, 0)),
        ],
        out_specs=pl.BlockSpec((QT, ho), lambda i: (i, 0)),
        out_shape=jax.ShapeDtypeStruct((nq, ho), jnp.float32),
    )(ug, t, w2, b2r, w3, b3r)


def kernel(x, pos, batch, W1, b1, W2, b2, W3, b3):
    n, f = x.shape
    bi = batch.astype(jnp.int32)
    offs = jnp.searchsorted(
        bi, jnp.arange(B + 1, dtype=jnp.int32), side="left").astype(jnp.int32)
    lo = offs[bi]
    hi = offs[bi + 1]
    posx = pos[:, 0]
    posy = pos[:, 1]
    posz = pos[:, 2]
    pos8 = jnp.pad(pos, ((0, 0), (0, 5)))
    hp = F - 64
    w1a = jnp.pad(W1[:f], ((0, 0), (0, hp)))
    w1b8 = jnp.pad(W1[f:], ((0, 5), (0, 0)))
    w1b8p = jnp.pad(w1b8, ((0, 0), (0, hp)))
    b1r = jnp.pad(b1, (0, hp)).reshape(1, -1)

    u2, t = _precompute(x, pos8, w1a, w1b8p, w1b8, b1r)
    b2r = b2.reshape(1, -1)
    b3r = b3.reshape(1, -1)
    nh = n // 2
    outs = []
    for qbase in (0, nh):
        ug = _sc_search_gather(posx, posy, posz, lo, hi, u2, qbase, nh)
        outs.append(_mlp(ug, t, W2, b2r, W3, b3r, qbase // QT))
    out = jnp.concatenate(outs, axis=0)
    return (out, pos, batch)
